# baseline (speedup 1.0000x reference)
import functools

import jax
import jax.numpy as jnp
from jax import lax
from jax.experimental import pallas as pl
from jax.experimental.pallas import tpu as pltpu
from jax.experimental.pallas import tpu_sc as plsc

N_ROWS = 320000
D = 128
S = 10000
CW = 8
CHUNK = 64
NC = 2
NS = 16
DH = D // NC
NCHUNKS = N_ROWS // CHUNK
BROWS = 2048
NG = 64
M_TC = NG * BROWS
W = 256
CHUNK0 = M_TC // CHUNK
FULL_ITERS = 184
REM = NCHUNKS - CHUNK0 - FULL_ITERS * NS
FULL_HOPS = S // CHUNK
TAIL = S - FULL_HOPS * CHUNK
HOPS_PER_TILE = 10
NBUF = 8


def _sc_segment_sum(x, segment_ids, zrow, zcnt, ones):
    mesh = plsc.VectorSubcoreMesh(core_axis_name="c", subcore_axis_name="s",
                                  num_cores=NC)

    @functools.partial(
        pl.kernel,
        out_type=(
            jax.ShapeDtypeStruct((S, D), jnp.float32),
            jax.ShapeDtypeStruct((NC, S, CW), jnp.float32),
        ),
        mesh=mesh,
        compiler_params=pltpu.CompilerParams(use_tc_tiling_on_sc=False),
        scratch_types=[
            pltpu.VMEM((NBUF, CHUNK), jnp.int32),
            pltpu.VMEM((NBUF, CHUNK, DH), jnp.float32),
            pltpu.VMEM((CHUNK, CW), jnp.float32),
            pltpu.VMEM_SHARED((S, DH), jnp.float32),
            pltpu.VMEM_SHARED((S, CW), jnp.float32),
        ] + [pltpu.SemaphoreType.DMA] * (3 * NBUF),
    )
    def k(x_hbm, ids_hbm, zrow_hbm, zcnt_hbm, ones_hbm,
          psum_hbm, pcnt_hbm,
          idx2, rows2, ones_v, accum, caccum, *sems):
        cid = lax.axis_index("c")
        sid = lax.axis_index("s")
        col0 = cid * DH
        sem_ld = sems[0:NBUF]
        sem_s = sems[NBUF:2 * NBUF]
        sem_c = sems[2 * NBUF:3 * NBUF]

        pltpu.sync_copy(zrow_hbm, rows2.at[0])
        pltpu.sync_copy(zcnt_hbm, ones_v)

        def hops(fn):
            for j in range(HOPS_PER_TILE):
                h = sid + j * NS

                @pl.when(h < FULL_HOPS)
                def _():
                    fn(h * CHUNK, CHUNK)

            @pl.when(sid == NS - 1)
            def _():
                fn(FULL_HOPS * CHUNK, TAIL)

        def init_hop(off, n):
            pltpu.sync_copy(rows2.at[0, pl.ds(0, n)],
                            accum.at[pl.ds(off, n)])
            pltpu.sync_copy(ones_v.at[pl.ds(0, n)],
                            caccum.at[pl.ds(off, n)])

        hops(init_hop)
        pltpu.sync_copy(ones_hbm, ones_v)
        plsc.subcore_barrier()

        def load_descs(k_idx, b):
            row0 = (CHUNK0 + sid + k_idx * NS) * CHUNK
            return (
                pltpu.make_async_copy(ids_hbm.at[pl.ds(row0, CHUNK)],
                                      idx2.at[b], sem_ld[b]),
                pltpu.make_async_copy(
                    x_hbm.at[pl.ds(row0, CHUNK), pl.ds(col0, DH)],
                    rows2.at[b], sem_ld[b]),
            )

        def issue_load(k_idx, b):
            di, dr = load_descs(k_idx, b)
            di.start()
            dr.start()

        def scatter_chunk(b):
            pltpu.async_copy(rows2.at[b], accum.at[idx2.at[b]],
                             sem_s[b], add=True)

            @pl.when(cid == b % 2)
            def _():
                pltpu.async_copy(ones_v, caccum.at[idx2.at[b]],
                                 sem_c[b], add=True)

        def drain_scatter(b):
            pltpu.make_async_copy(rows2.at[b], accum.at[idx2.at[b]],
                                  sem_s[b]).wait()

            @pl.when(cid == b % 2)
            def _():
                pltpu.make_async_copy(ones_v, caccum.at[idx2.at[b]],
                                      sem_c[b]).wait()

        for b in range(NBUF - 1):
            issue_load(b, b)

        def step(i, _):
            for b in range(NBUF):
                k_idx = i * NBUF + b
                di, dr = load_descs(k_idx, b)
                di.wait()
                dr.wait()
                scatter_chunk(b)

                pb = (b - 1) % NBUF

                @pl.when(k_idx >= 1)
                def _():
                    drain_scatter(pb)

                @pl.when(k_idx + NBUF - 1 < FULL_ITERS)
                def _():
                    issue_load(k_idx + NBUF - 1, pb)
            return 0

        lax.fori_loop(0, FULL_ITERS // NBUF, step, 0)
        drain_scatter((FULL_ITERS - 1) % NBUF)

        @pl.when(sid < REM)
        def _():
            row0 = (CHUNK0 + FULL_ITERS * NS + sid) * CHUNK
            pltpu.sync_copy(ids_hbm.at[pl.ds(row0, CHUNK)], idx2.at[0])
            pltpu.sync_copy(x_hbm.at[pl.ds(row0, CHUNK), pl.ds(col0, DH)],
                            rows2.at[0])
            pltpu.sync_copy(rows2.at[0], accum.at[idx2.at[0]], add=True)

            @pl.when(cid == sid % 2)
            def _():
                pltpu.sync_copy(ones_v, caccum.at[idx2.at[0]], add=True)

        plsc.subcore_barrier()

        def export_hop(off, n):
            pltpu.sync_copy(accum.at[pl.ds(off, n)],
                            rows2.at[0, pl.ds(0, n)])
            pltpu.sync_copy(rows2.at[0, pl.ds(0, n)],
                            psum_hbm.at[pl.ds(off, n), pl.ds(col0, DH)])

            pltpu.sync_copy(caccum.at[pl.ds(off, n)],
                            ones_v.at[pl.ds(0, n)])
            pltpu.sync_copy(ones_v.at[pl.ds(0, n)],
                            pcnt_hbm.at[cid, pl.ds(off, n)])

        hops(export_hop)

    return k(x, segment_ids, zrow, zcnt, ones)


def _tc_onehot_body(bases_sref, ids_ref, x_ref, psum_ref, cnt_ref,
                    acc, cacc):
    g = pl.program_id(0)

    @pl.when(g == 0)
    def _():
        acc[...] = jnp.zeros_like(acc)
        cacc[...] = jnp.zeros_like(cacc)

    base = pl.multiple_of(bases_sref[g], 8)
    idsb = ids_ref[...]
    seg = base + lax.broadcasted_iota(jnp.int32, (W, BROWS), 0)
    mask = jnp.where(seg == idsb, 1.0, 0.0)
    blk = jnp.dot(mask, x_ref[...], preferred_element_type=jnp.float32)
    acc[pl.ds(base, W), :] += blk
    cacc[pl.ds(base, W), :] += jnp.sum(mask, axis=1, keepdims=True)

    @pl.when(g == NG - 1)
    def _():
        psum_ref[...] = acc[...]
        cnt_ref[...] = cacc[...]


def _tc_segment_sum(x, segment_ids):
    bases = segment_ids[::BROWS][:NG]
    bases = jnp.minimum(bases - (bases % 8), S - W).astype(jnp.int32)
    ids_row = segment_ids.reshape(1, N_ROWS)

    grid_spec = pltpu.PrefetchScalarGridSpec(
        num_scalar_prefetch=1,
        grid=(NG,),
        in_specs=[
            pl.BlockSpec((1, BROWS), lambda g, bases_ref: (0, g)),
            pl.BlockSpec((BROWS, D), lambda g, bases_ref: (g, 0)),
        ],
        out_specs=[
            pl.BlockSpec((S, D), lambda g, bases_ref: (0, 0)),
            pl.BlockSpec((S, 1), lambda g, bases_ref: (0, 0)),
        ],
        scratch_shapes=[
            pltpu.VMEM((S, D), jnp.float32),
            pltpu.VMEM((S, 1), jnp.float32),
        ],
    )
    return pl.pallas_call(
        _tc_onehot_body,
        grid_spec=grid_spec,
        out_shape=(
            jax.ShapeDtypeStruct((S, D), jnp.float32),
            jax.ShapeDtypeStruct((S, 1), jnp.float32),
        ),
    )(bases, ids_row, x)


def _tc_mlp_body(psum_ref, pcnt_ref, tsum_ref, tcnt_ref,
                 w1_ref, b1_ref, w2_ref, b2_ref, w3_ref, b3_ref, out_ref):
    cnt = pcnt_ref[0, :, 0:1] + pcnt_ref[1, :, 0:1] + tcnt_ref[...]
    pooled = (psum_ref[...] + tsum_ref[...]) / jnp.maximum(cnt, 1.0)
    h = jnp.maximum(jnp.dot(pooled, w1_ref[...],
                            preferred_element_type=jnp.float32)
                    + b1_ref[...], 0.0)
    h = jnp.maximum(jnp.dot(h, w2_ref[...],
                            preferred_element_type=jnp.float32)
                    + b2_ref[...], 0.0)
    out_ref[...] = (jnp.dot(h, w3_ref[...], preferred_element_type=jnp.float32)
                    + b3_ref[...])


def kernel(x, segment_ids, W1, b1, W2, b2, W3, b3):
    zrow = jnp.zeros((CHUNK, DH), jnp.float32)
    zcnt = jnp.zeros((CHUNK, CW), jnp.float32)
    ones = jnp.ones((CHUNK, CW), jnp.float32)
    psum, pcnt = _sc_segment_sum(x, segment_ids, zrow, zcnt, ones)
    tsum, tcnt = _tc_segment_sum(x, segment_ids)

    n_out = W3.shape[1]
    out = pl.pallas_call(
        _tc_mlp_body,
        out_shape=jax.ShapeDtypeStruct((S, n_out), jnp.float32),
    )(psum, pcnt, tsum, tcnt, W1, b1.reshape(1, -1), W2, b2.reshape(1, -1),
      W3, b3.reshape(1, -1))
    return out


# speedup vs baseline: 1.0471x; 1.0471x over previous
import functools

import jax
import jax.numpy as jnp
from jax import lax
from jax.experimental import pallas as pl
from jax.experimental.pallas import tpu as pltpu
from jax.experimental.pallas import tpu_sc as plsc

N_ROWS = 320000
D = 128
S = 10000
CW = 8
CHUNK = 64
NC = 2
NS = 16
DH = D // NC
NCHUNKS = N_ROWS // CHUNK
BROWS = 2048
NG = 64
M_TC = NG * BROWS
W = 256
CHUNK0 = M_TC // CHUNK
FULL_ITERS = 184
REM = NCHUNKS - CHUNK0 - FULL_ITERS * NS
FULL_HOPS = S // CHUNK
TAIL = S - FULL_HOPS * CHUNK
HOPS_PER_TILE = 10
NBUF = 8


def _sc_segment_sum(x, segment_ids, zrow, zcnt, ones):
    mesh = plsc.VectorSubcoreMesh(core_axis_name="c", subcore_axis_name="s",
                                  num_cores=NC)

    @functools.partial(
        pl.kernel,
        out_type=(
            jax.ShapeDtypeStruct((S, D), jnp.float32),
            jax.ShapeDtypeStruct((S, CW), jnp.float32),
        ),
        mesh=mesh,
        compiler_params=pltpu.CompilerParams(use_tc_tiling_on_sc=False),
        scratch_types=[
            pltpu.VMEM((NBUF, CHUNK), jnp.int32),
            pltpu.VMEM((NBUF, CHUNK, DH), jnp.float32),
            pltpu.VMEM((CHUNK, CW), jnp.float32),
            pltpu.VMEM_SHARED((S, DH), jnp.float32),
            pltpu.VMEM_SHARED((S, CW), jnp.float32),
        ] + [pltpu.SemaphoreType.DMA] * (3 * NBUF),
    )
    def k(x_hbm, ids_hbm, zrow_hbm, zcnt_hbm, ones_hbm,
          psum_hbm, pcnt_hbm,
          idx2, rows2, ones_v, accum, caccum, *sems):
        cid = lax.axis_index("c")
        sid = lax.axis_index("s")
        col0 = cid * DH
        sem_ld = sems[0:NBUF]
        sem_s = sems[NBUF:2 * NBUF]
        sem_c = sems[2 * NBUF:3 * NBUF]

        pltpu.sync_copy(zrow_hbm, rows2.at[0])
        pltpu.sync_copy(zcnt_hbm, ones_v)

        def hops(fn):
            for j in range(HOPS_PER_TILE):
                h = sid + j * NS

                @pl.when(h < FULL_HOPS)
                def _():
                    fn(h * CHUNK, CHUNK)

            @pl.when(sid == NS - 1)
            def _():
                fn(FULL_HOPS * CHUNK, TAIL)

        def init_hop(off, n):
            pltpu.sync_copy(rows2.at[0, pl.ds(0, n)],
                            accum.at[pl.ds(off, n)])
            pltpu.sync_copy(ones_v.at[pl.ds(0, n)],
                            caccum.at[pl.ds(off, n)])

        hops(init_hop)
        pltpu.sync_copy(ones_hbm, ones_v)
        plsc.subcore_barrier()

        def load_descs(k_idx, b):
            row0 = (CHUNK0 + sid + k_idx * NS) * CHUNK
            return (
                pltpu.make_async_copy(ids_hbm.at[pl.ds(row0, CHUNK)],
                                      idx2.at[b], sem_ld[b]),
                pltpu.make_async_copy(
                    x_hbm.at[pl.ds(row0, CHUNK), pl.ds(col0, DH)],
                    rows2.at[b], sem_ld[b]),
            )

        def issue_load(k_idx, b):
            di, dr = load_descs(k_idx, b)
            di.start()
            dr.start()

        def scatter_chunk(b):
            pltpu.async_copy(rows2.at[b], accum.at[idx2.at[b]],
                             sem_s[b], add=True)

            @pl.when(cid == 0)
            def _():
                pltpu.async_copy(ones_v, caccum.at[idx2.at[b]],
                                 sem_c[b], add=True)

        def drain_scatter(b):
            pltpu.make_async_copy(rows2.at[b], accum.at[idx2.at[b]],
                                  sem_s[b]).wait()

            @pl.when(cid == 0)
            def _():
                pltpu.make_async_copy(ones_v, caccum.at[idx2.at[b]],
                                      sem_c[b]).wait()

        for b in range(NBUF - 1):
            issue_load(b, b)

        def step(i, _):
            for b in range(NBUF):
                k_idx = i * NBUF + b
                di, dr = load_descs(k_idx, b)
                di.wait()
                dr.wait()
                scatter_chunk(b)

                pb = (b - 1) % NBUF

                @pl.when(k_idx >= 1)
                def _():
                    drain_scatter(pb)

                @pl.when(k_idx + NBUF - 1 < FULL_ITERS)
                def _():
                    issue_load(k_idx + NBUF - 1, pb)
            return 0

        lax.fori_loop(0, FULL_ITERS // NBUF, step, 0)
        drain_scatter((FULL_ITERS - 1) % NBUF)

        @pl.when(sid < REM)
        def _():
            row0 = (CHUNK0 + FULL_ITERS * NS + sid) * CHUNK
            pltpu.sync_copy(ids_hbm.at[pl.ds(row0, CHUNK)], idx2.at[0])
            pltpu.sync_copy(x_hbm.at[pl.ds(row0, CHUNK), pl.ds(col0, DH)],
                            rows2.at[0])
            pltpu.sync_copy(rows2.at[0], accum.at[idx2.at[0]], add=True)

            @pl.when(cid == 0)
            def _():
                pltpu.sync_copy(ones_v, caccum.at[idx2.at[0]], add=True)

        plsc.subcore_barrier()

        def export_hop(off, n):
            pltpu.sync_copy(accum.at[pl.ds(off, n)],
                            rows2.at[0, pl.ds(0, n)])
            pltpu.sync_copy(rows2.at[0, pl.ds(0, n)],
                            psum_hbm.at[pl.ds(off, n), pl.ds(col0, DH)])

            @pl.when(cid == 0)
            def _():
                pltpu.sync_copy(caccum.at[pl.ds(off, n)],
                                ones_v.at[pl.ds(0, n)])
                pltpu.sync_copy(ones_v.at[pl.ds(0, n)],
                                pcnt_hbm.at[pl.ds(off, n)])

        hops(export_hop)

    return k(x, segment_ids, zrow, zcnt, ones)


def _tc_onehot_body(bases_sref, ids_ref, x_ref, psum_ref, cnt_ref,
                    acc, cacc):
    g = pl.program_id(0)

    @pl.when(g == 0)
    def _():
        acc[...] = jnp.zeros_like(acc)
        cacc[...] = jnp.zeros_like(cacc)

    base = pl.multiple_of(bases_sref[g], 8)
    idsb = ids_ref[...]
    seg = base + lax.broadcasted_iota(jnp.int32, (W, BROWS), 0)
    mask = jnp.where(seg == idsb, 1.0, 0.0)
    blk = jnp.dot(mask, x_ref[...], preferred_element_type=jnp.float32)
    acc[pl.ds(base, W), :] += blk
    cacc[pl.ds(base, W), :] += jnp.sum(mask, axis=1, keepdims=True)

    @pl.when(g == NG - 1)
    def _():
        psum_ref[...] = acc[...]
        cnt_ref[...] = cacc[...]


def _tc_segment_sum(x, segment_ids):
    bases = segment_ids[::BROWS][:NG]
    bases = jnp.minimum(bases - (bases % 8), S - W).astype(jnp.int32)
    ids_row = segment_ids.reshape(1, N_ROWS)

    grid_spec = pltpu.PrefetchScalarGridSpec(
        num_scalar_prefetch=1,
        grid=(NG,),
        in_specs=[
            pl.BlockSpec((1, BROWS), lambda g, bases_ref: (0, g)),
            pl.BlockSpec((BROWS, D), lambda g, bases_ref: (g, 0)),
        ],
        out_specs=[
            pl.BlockSpec((S, D), lambda g, bases_ref: (0, 0)),
            pl.BlockSpec((S, 1), lambda g, bases_ref: (0, 0)),
        ],
        scratch_shapes=[
            pltpu.VMEM((S, D), jnp.float32),
            pltpu.VMEM((S, 1), jnp.float32),
        ],
    )
    return pl.pallas_call(
        _tc_onehot_body,
        grid_spec=grid_spec,
        out_shape=(
            jax.ShapeDtypeStruct((S, D), jnp.float32),
            jax.ShapeDtypeStruct((S, 1), jnp.float32),
        ),
    )(bases, ids_row, x)


def _tc_mlp_body(psum_ref, pcnt_ref, tsum_ref, tcnt_ref,
                 w1_ref, b1_ref, w2_ref, b2_ref, w3_ref, b3_ref, out_ref):
    cnt = pcnt_ref[:, 0:1] + tcnt_ref[...]
    pooled = (psum_ref[...] + tsum_ref[...]) / jnp.maximum(cnt, 1.0)
    h = jnp.maximum(jnp.dot(pooled, w1_ref[...],
                            preferred_element_type=jnp.float32)
                    + b1_ref[...], 0.0)
    h = jnp.maximum(jnp.dot(h, w2_ref[...],
                            preferred_element_type=jnp.float32)
                    + b2_ref[...], 0.0)
    out_ref[...] = (jnp.dot(h, w3_ref[...], preferred_element_type=jnp.float32)
                    + b3_ref[...])


def kernel(x, segment_ids, W1, b1, W2, b2, W3, b3):
    zrow = jnp.zeros((CHUNK, DH), jnp.float32)
    zcnt = jnp.zeros((CHUNK, CW), jnp.float32)
    ones = jnp.ones((CHUNK, CW), jnp.float32)
    psum, pcnt = _sc_segment_sum(x, segment_ids, zrow, zcnt, ones)
    tsum, tcnt = _tc_segment_sum(x, segment_ids)

    n_out = W3.shape[1]
    out = pl.pallas_call(
        _tc_mlp_body,
        out_shape=jax.ShapeDtypeStruct((S, n_out), jnp.float32),
    )(psum, pcnt, tsum, tcnt, W1, b1.reshape(1, -1), W2, b2.reshape(1, -1),
      W3, b3.reshape(1, -1))
    return out
